# Initial kernel scaffold; baseline (speedup 1.0000x reference)
import jax
import jax.numpy as jnp
from jax.experimental import pallas as pl


def kernel(preds, labels, weights):
    raise NotImplementedError("write your pallas kernel here")


# speedup vs baseline: 12.4859x; 12.4859x over previous
import functools

import jax
import jax.numpy as jnp
from jax import lax
from jax.experimental import pallas as pl
from jax.experimental.pallas import tpu as pltpu
from jax.experimental.pallas import tpu_sc as plsc

IGNORE = 255
THRESH = 0.35667494393873245

B, C, H, W = 8, 19, 512, 512
HW = H * W
NPIX = B * HW
NC, NS = 2, 16
NTILES = NC * NS
PIX_PER_TILE = NPIX // NTILES
P = 2048
NCHUNK = PIX_PER_TILE // P
NGRP = P // 16
Q_PER_IMG = HW // PIX_PER_TILE

_mesh = plsc.VectorSubcoreMesh(core_axis_name="c", subcore_axis_name="s")
_cparams = pltpu.CompilerParams(needs_layout_passes=False,
                                use_tc_tiling_on_sc=False)


def _copy_chunk_in(preds_hbm, labels_hbm, weights_hbm, b, p0,
                   pred_v, lab_v, wgt_v, sem):
    copies = [pltpu.async_copy(preds_hbm.at[pl.ds(b * (C * HW) + c * HW + p0, P)],
                               pred_v.at[pl.ds(c * P, P)], sem)
              for c in range(C)]
    copies.append(pltpu.async_copy(labels_hbm.at[pl.ds(b * HW + p0, P)],
                                   lab_v, sem))
    copies.append(pltpu.async_copy(weights_hbm.at[pl.ds(b * HW + p0, P)],
                                   wgt_v, sem))
    for cp in copies:
        cp.wait()


def _loss_group(pred_v, lab_v, wgt_v, g):
    lab = lab_v[pl.ds(g * 16, 16)]
    wgt = wgt_v[pl.ds(g * 16, 16)]
    valid = lab != IGNORE
    safe_lab = jnp.where(valid, lab, 0)
    pos = g * 16 + lax.iota(jnp.int32, 16)
    val = plsc.load_gather(pred_v, [safe_lab * P + pos])
    loss = jnp.where(valid, wgt * (-val), jnp.zeros((16,), jnp.float32))
    return loss, valid


@functools.partial(
    pl.kernel,
    out_type=(
        jax.ShapeDtypeStruct((NTILES * 16,), jnp.float32),
        jax.ShapeDtypeStruct((NTILES * 16,), jnp.int32),
        jax.ShapeDtypeStruct((NTILES * 16,), jnp.int32),
    ),
    mesh=_mesh,
    scratch_types=(
        pltpu.VMEM((C * P,), jnp.float32),
        pltpu.VMEM((P,), jnp.int32),
        pltpu.VMEM((P,), jnp.float32),
        pltpu.VMEM((16,), jnp.float32),
        pltpu.VMEM((16,), jnp.int32),
        pltpu.VMEM((16,), jnp.int32),
        pltpu.SemaphoreType.DMA,
    ),
    compiler_params=_cparams,
)
def _main_kernel(preds_hbm, labels_hbm, weights_hbm,
                 hsum_out, hcnt_out, vcnt_out,
                 pred_v, lab_v, wgt_v, hs_v, hc_v, vc_v, sem):
    wid = lax.axis_index("s") * NC + lax.axis_index("c")
    b = wid // Q_PER_IMG
    q = wid % Q_PER_IMG

    def chunk_body(i, carry):
        hs, hc, vc = carry
        p0 = q * PIX_PER_TILE + i * P
        _copy_chunk_in(preds_hbm, labels_hbm, weights_hbm, b, p0,
                       pred_v, lab_v, wgt_v, sem)

        def g_body(g, inner):
            hs, hc, vc = inner
            loss, valid = _loss_group(pred_v, lab_v, wgt_v, g)
            hard = loss > THRESH
            hs = hs + jnp.where(hard, loss, jnp.zeros((16,), jnp.float32))
            hc = hc + jnp.where(hard, jnp.ones((16,), jnp.int32),
                                jnp.zeros((16,), jnp.int32))
            vc = vc + jnp.where(valid, jnp.ones((16,), jnp.int32),
                                jnp.zeros((16,), jnp.int32))
            return hs, hc, vc

        return lax.fori_loop(0, NGRP, g_body, (hs, hc, vc))

    hs, hc, vc = lax.fori_loop(
        0, NCHUNK, chunk_body,
        (jnp.zeros((16,), jnp.float32), jnp.zeros((16,), jnp.int32),
         jnp.zeros((16,), jnp.int32)))
    hs_v[...] = hs
    hc_v[...] = hc
    vc_v[...] = vc
    pltpu.sync_copy(hs_v, hsum_out.at[pl.ds(wid * 16, 16)])
    pltpu.sync_copy(hc_v, hcnt_out.at[pl.ds(wid * 16, 16)])
    pltpu.sync_copy(vc_v, vcnt_out.at[pl.ds(wid * 16, 16)])


@functools.partial(
    pl.kernel,
    out_type=(
        jax.ShapeDtypeStruct((NTILES * 16,), jnp.int32),
        jax.ShapeDtypeStruct((NTILES * 16,), jnp.int32),
        jax.ShapeDtypeStruct((NTILES * 16,), jnp.float32),
    ),
    mesh=_mesh,
    scratch_types=(
        pltpu.VMEM((C * P,), jnp.float32),
        pltpu.VMEM((P,), jnp.int32),
        pltpu.VMEM((P,), jnp.float32),
        pltpu.VMEM((16,), jnp.uint32),
        pltpu.VMEM((16,), jnp.float32),
        pltpu.VMEM((16,), jnp.int32),
        pltpu.VMEM((16,), jnp.int32),
        pltpu.VMEM((16,), jnp.float32),
        pltpu.SemaphoreType.DMA,
    ),
    compiler_params=_cparams,
)
def _stats_kernel(preds_hbm, labels_hbm, weights_hbm, yu_hbm, tf_hbm,
                  ucnt_out, gcnt_out, gsum_out,
                  pred_v, lab_v, wgt_v, yu_v, tf_v, uc_v, gc_v, gs_v, sem):
    wid = lax.axis_index("s") * NC + lax.axis_index("c")
    b = wid // Q_PER_IMG
    q = wid % Q_PER_IMG
    pltpu.sync_copy(yu_hbm, yu_v)
    pltpu.sync_copy(tf_hbm, tf_v)
    yu = yu_v[...]
    tf = tf_v[...]

    def chunk_body(i, carry):
        uc, gc, gs = carry
        p0 = q * PIX_PER_TILE + i * P
        _copy_chunk_in(preds_hbm, labels_hbm, weights_hbm, b, p0,
                       pred_v, lab_v, wgt_v, sem)

        def g_body(g, inner):
            uc, gc, gs = inner
            loss, _ = _loss_group(pred_v, lab_v, wgt_v, g)
            bits = plsc.bitcast(loss, jnp.uint32)
            sign = bits >> jnp.uint32(31)
            flip = (jnp.zeros((16,), jnp.uint32) - sign) | jnp.uint32(0x80000000)
            u = bits ^ flip
            uc = uc + jnp.where(u >= yu, jnp.ones((16,), jnp.int32),
                                jnp.zeros((16,), jnp.int32))
            gt = loss > tf
            gc = gc + jnp.where(gt, jnp.ones((16,), jnp.int32),
                                jnp.zeros((16,), jnp.int32))
            gs = gs + jnp.where(gt, loss, jnp.zeros((16,), jnp.float32))
            return uc, gc, gs

        return lax.fori_loop(0, NGRP, g_body, (uc, gc, gs))

    uc, gc, gs = lax.fori_loop(
        0, NCHUNK, chunk_body,
        (jnp.zeros((16,), jnp.int32), jnp.zeros((16,), jnp.int32),
         jnp.zeros((16,), jnp.float32)))
    uc_v[...] = uc
    gc_v[...] = gc
    gs_v[...] = gs
    pltpu.sync_copy(uc_v, ucnt_out.at[pl.ds(wid * 16, 16)])
    pltpu.sync_copy(gc_v, gcnt_out.at[pl.ds(wid * 16, 16)])
    pltpu.sync_copy(gs_v, gsum_out.at[pl.ds(wid * 16, 16)])


def kernel(preds, labels, weights):
    preds3 = preds.reshape(B * C * HW)
    labels2 = labels.reshape(B * HW)
    weights2 = weights.reshape(B * HW)

    hsum, hcnt, vcnt = _main_kernel(preds3, labels2, weights2)
    sum_hard = jnp.sum(hsum)
    n_hard = jnp.sum(hcnt)
    n_valid = jnp.sum(vcnt)
    n_min = n_valid // 16

    def fast_path(_):
        return sum_hard / jnp.maximum(n_hard, 1).astype(jnp.float32)

    def slow_path(_):
        def bit_body(i, x):
            y = x | (jnp.uint32(1) << (jnp.uint32(31) - i.astype(jnp.uint32)))
            yv = jnp.full((16,), y, jnp.uint32)
            tv = jnp.zeros((16,), jnp.float32)
            ucnt, _, _ = _stats_kernel(preds3, labels2, weights2, yv, tv)
            return jnp.where(jnp.sum(ucnt) >= n_min, y, x)

        t_u = lax.fori_loop(0, 32, bit_body, jnp.uint32(0))
        t_bits = jnp.where(t_u >= jnp.uint32(0x80000000),
                           t_u ^ jnp.uint32(0x80000000), ~t_u)
        t = lax.bitcast_convert_type(t_bits, jnp.float32)
        yv = jnp.full((16,), t_u, jnp.uint32)
        tv = jnp.full((16,), t, jnp.float32)
        _, gcnt, gsum = _stats_kernel(preds3, labels2, weights2, yv, tv)
        c_gt = jnp.sum(gcnt)
        s_gt = jnp.sum(gsum)
        kf = n_min.astype(jnp.float32)
        return (s_gt + (n_min - c_gt).astype(jnp.float32) * t) / kf

    return lax.cond(n_hard >= n_min, fast_path, slow_path, operand=None)


# speedup vs baseline: 13.3471x; 1.0690x over previous
import functools

import jax
import jax.numpy as jnp
from jax import lax
from jax.experimental import pallas as pl
from jax.experimental.pallas import tpu as pltpu
from jax.experimental.pallas import tpu_sc as plsc

IGNORE = 255
THRESH = 0.35667494393873245

B, C, H, W = 8, 19, 512, 512
HW = H * W
NPIX = B * HW
NC, NS = 2, 16
NTILES = NC * NS
PIX_PER_TILE = NPIX // NTILES
P = 2048
NCHUNK = PIX_PER_TILE // P
NGRP = P // 16
Q_PER_IMG = HW // PIX_PER_TILE

_mesh = plsc.VectorSubcoreMesh(core_axis_name="c", subcore_axis_name="s")
_cparams = pltpu.CompilerParams(needs_layout_passes=False,
                                use_tc_tiling_on_sc=False)


def _copy_chunk_in(preds_hbm, labels_hbm, weights_hbm, b, p0,
                   pred_v, lab_v, wgt_v, sem):
    copies = [pltpu.async_copy(preds_hbm.at[pl.ds(b * (C * HW) + c * HW + p0, P)],
                               pred_v.at[pl.ds(c * P, P)], sem)
              for c in range(C)]
    copies.append(pltpu.async_copy(labels_hbm.at[pl.ds(b * HW + p0, P)],
                                   lab_v, sem))
    copies.append(pltpu.async_copy(weights_hbm.at[pl.ds(b * HW + p0, P)],
                                   wgt_v, sem))
    for cp in copies:
        cp.wait()


def _loss_group(pred_v, lab_v, wgt_v, g):
    lab = lab_v[pl.ds(g * 16, 16)]
    wgt = wgt_v[pl.ds(g * 16, 16)]
    valid = lab != IGNORE
    safe_lab = jnp.where(valid, lab, 0)
    pos = g * 16 + lax.iota(jnp.int32, 16)
    val = plsc.load_gather(pred_v, [safe_lab * P + pos])
    loss = jnp.where(valid, wgt * (-val), jnp.zeros((16,), jnp.float32))
    return loss, valid


@functools.partial(
    pl.kernel,
    out_type=(
        jax.ShapeDtypeStruct((NTILES * 16,), jnp.float32),
        jax.ShapeDtypeStruct((NTILES * 16,), jnp.int32),
        jax.ShapeDtypeStruct((NTILES * 16,), jnp.int32),
    ),
    mesh=_mesh,
    scratch_types=(
        pltpu.VMEM((C * P,), jnp.float32),
        pltpu.VMEM((C * P,), jnp.float32),
        pltpu.VMEM((P,), jnp.int32),
        pltpu.VMEM((P,), jnp.int32),
        pltpu.VMEM((P,), jnp.float32),
        pltpu.VMEM((P,), jnp.float32),
        pltpu.VMEM((16,), jnp.float32),
        pltpu.VMEM((16,), jnp.int32),
        pltpu.VMEM((16,), jnp.int32),
        pltpu.SemaphoreType.DMA,
        pltpu.SemaphoreType.DMA,
    ),
    compiler_params=_cparams,
)
def _main_kernel(preds_hbm, labels_hbm, weights_hbm,
                 hsum_out, hcnt_out, vcnt_out,
                 pred_v0, pred_v1, lab_v0, lab_v1, wgt_v0, wgt_v1,
                 hs_v, hc_v, vc_v, sem0, sem1):
    wid = lax.axis_index("s") * NC + lax.axis_index("c")
    b = wid // Q_PER_IMG
    q = wid % Q_PER_IMG

    def start_chunk(i, pv, lv, wv, sem):
        p0 = q * PIX_PER_TILE + i * P
        for c in range(C):
            pltpu.async_copy(
                preds_hbm.at[pl.ds(b * (C * HW) + c * HW + p0, P)],
                pv.at[pl.ds(c * P, P)], sem)
        pltpu.async_copy(labels_hbm.at[pl.ds(b * HW + p0, P)], lv, sem)
        pltpu.async_copy(weights_hbm.at[pl.ds(b * HW + p0, P)], wv, sem)

    def wait_chunk(pv, lv, wv, sem):
        pltpu.make_async_copy(preds_hbm.at[pl.ds(0, C * P)], pv, sem).wait()
        pltpu.make_async_copy(labels_hbm.at[pl.ds(0, P)], lv, sem).wait()
        pltpu.make_async_copy(weights_hbm.at[pl.ds(0, P)], wv, sem).wait()

    def compute(pv, lv, wv, carry):
        def g_body(g, inner):
            hs, hc, vc = inner
            loss, valid = _loss_group(pv, lv, wv, g)
            hard = loss > THRESH
            hs = hs + jnp.where(hard, loss, jnp.zeros((16,), jnp.float32))
            hc = hc + hard.astype(jnp.int32)
            vc = vc + valid.astype(jnp.int32)
            return hs, hc, vc

        return lax.fori_loop(0, NGRP, g_body, carry, unroll=8)

    start_chunk(0, pred_v0, lab_v0, wgt_v0, sem0)

    def pair_body(i2, carry):
        start_chunk(2 * i2 + 1, pred_v1, lab_v1, wgt_v1, sem1)
        wait_chunk(pred_v0, lab_v0, wgt_v0, sem0)
        carry = compute(pred_v0, lab_v0, wgt_v0, carry)
        nxt = jnp.minimum(2 * i2 + 2, NCHUNK - 1)
        start_chunk(nxt, pred_v0, lab_v0, wgt_v0, sem0)
        wait_chunk(pred_v1, lab_v1, wgt_v1, sem1)
        carry = compute(pred_v1, lab_v1, wgt_v1, carry)
        return carry

    hs, hc, vc = lax.fori_loop(
        0, NCHUNK // 2, pair_body,
        (jnp.zeros((16,), jnp.float32), jnp.zeros((16,), jnp.int32),
         jnp.zeros((16,), jnp.int32)))
    wait_chunk(pred_v0, lab_v0, wgt_v0, sem0)
    hs_v[...] = hs
    hc_v[...] = hc
    vc_v[...] = vc
    pltpu.sync_copy(hs_v, hsum_out.at[pl.ds(wid * 16, 16)])
    pltpu.sync_copy(hc_v, hcnt_out.at[pl.ds(wid * 16, 16)])
    pltpu.sync_copy(vc_v, vcnt_out.at[pl.ds(wid * 16, 16)])


@functools.partial(
    pl.kernel,
    out_type=(
        jax.ShapeDtypeStruct((NTILES * 16,), jnp.int32),
        jax.ShapeDtypeStruct((NTILES * 16,), jnp.int32),
        jax.ShapeDtypeStruct((NTILES * 16,), jnp.float32),
    ),
    mesh=_mesh,
    scratch_types=(
        pltpu.VMEM((C * P,), jnp.float32),
        pltpu.VMEM((P,), jnp.int32),
        pltpu.VMEM((P,), jnp.float32),
        pltpu.VMEM((16,), jnp.uint32),
        pltpu.VMEM((16,), jnp.float32),
        pltpu.VMEM((16,), jnp.int32),
        pltpu.VMEM((16,), jnp.int32),
        pltpu.VMEM((16,), jnp.float32),
        pltpu.SemaphoreType.DMA,
    ),
    compiler_params=_cparams,
)
def _stats_kernel(preds_hbm, labels_hbm, weights_hbm, yu_hbm, tf_hbm,
                  ucnt_out, gcnt_out, gsum_out,
                  pred_v, lab_v, wgt_v, yu_v, tf_v, uc_v, gc_v, gs_v, sem):
    wid = lax.axis_index("s") * NC + lax.axis_index("c")
    b = wid // Q_PER_IMG
    q = wid % Q_PER_IMG
    pltpu.sync_copy(yu_hbm, yu_v)
    pltpu.sync_copy(tf_hbm, tf_v)
    yu = yu_v[...]
    tf = tf_v[...]

    def chunk_body(i, carry):
        uc, gc, gs = carry
        p0 = q * PIX_PER_TILE + i * P
        _copy_chunk_in(preds_hbm, labels_hbm, weights_hbm, b, p0,
                       pred_v, lab_v, wgt_v, sem)

        def g_body(g, inner):
            uc, gc, gs = inner
            loss, _ = _loss_group(pred_v, lab_v, wgt_v, g)
            bits = plsc.bitcast(loss, jnp.uint32)
            sign = bits >> jnp.uint32(31)
            flip = (jnp.zeros((16,), jnp.uint32) - sign) | jnp.uint32(0x80000000)
            u = bits ^ flip
            uc = uc + jnp.where(u >= yu, jnp.ones((16,), jnp.int32),
                                jnp.zeros((16,), jnp.int32))
            gt = loss > tf
            gc = gc + jnp.where(gt, jnp.ones((16,), jnp.int32),
                                jnp.zeros((16,), jnp.int32))
            gs = gs + jnp.where(gt, loss, jnp.zeros((16,), jnp.float32))
            return uc, gc, gs

        return lax.fori_loop(0, NGRP, g_body, (uc, gc, gs))

    uc, gc, gs = lax.fori_loop(
        0, NCHUNK, chunk_body,
        (jnp.zeros((16,), jnp.int32), jnp.zeros((16,), jnp.int32),
         jnp.zeros((16,), jnp.float32)))
    uc_v[...] = uc
    gc_v[...] = gc
    gs_v[...] = gs
    pltpu.sync_copy(uc_v, ucnt_out.at[pl.ds(wid * 16, 16)])
    pltpu.sync_copy(gc_v, gcnt_out.at[pl.ds(wid * 16, 16)])
    pltpu.sync_copy(gs_v, gsum_out.at[pl.ds(wid * 16, 16)])


def kernel(preds, labels, weights):
    preds3 = preds.reshape(B * C * HW)
    labels2 = labels.reshape(B * HW)
    weights2 = weights.reshape(B * HW)

    hsum, hcnt, vcnt = _main_kernel(preds3, labels2, weights2)
    sum_hard = jnp.sum(hsum)
    n_hard = jnp.sum(hcnt)
    n_valid = jnp.sum(vcnt)
    n_min = n_valid // 16

    def fast_path(_):
        return sum_hard / jnp.maximum(n_hard, 1).astype(jnp.float32)

    def slow_path(_):
        def bit_body(i, x):
            y = x | (jnp.uint32(1) << (jnp.uint32(31) - i.astype(jnp.uint32)))
            yv = jnp.full((16,), y, jnp.uint32)
            tv = jnp.zeros((16,), jnp.float32)
            ucnt, _, _ = _stats_kernel(preds3, labels2, weights2, yv, tv)
            return jnp.where(jnp.sum(ucnt) >= n_min, y, x)

        t_u = lax.fori_loop(0, 32, bit_body, jnp.uint32(0))
        t_bits = jnp.where(t_u >= jnp.uint32(0x80000000),
                           t_u ^ jnp.uint32(0x80000000), ~t_u)
        t = lax.bitcast_convert_type(t_bits, jnp.float32)
        yv = jnp.full((16,), t_u, jnp.uint32)
        tv = jnp.full((16,), t, jnp.float32)
        _, gcnt, gsum = _stats_kernel(preds3, labels2, weights2, yv, tv)
        c_gt = jnp.sum(gcnt)
        s_gt = jnp.sum(gsum)
        kf = n_min.astype(jnp.float32)
        return (s_gt + (n_min - c_gt).astype(jnp.float32) * t) / kf

    return lax.cond(n_hard >= n_min, fast_path, slow_path, operand=None)
